# baseline (speedup 1.0000x reference)
import jax
import jax.numpy as jnp
from jax import lax
from jax.experimental import pallas as pl
from jax.experimental.pallas import tpu as pltpu
from jax.experimental.pallas import tpu_sc as plsc

DIM = 128
CAP = 100000

BLK = 2048
NBLK = 45
TC_ROWS = NBLK * BLK
SC_START = TC_ROWS
SC_ROWS = CAP - TC_ROWS

NC = 1
NS = 16
NW = NC * NS
LANES = 16
CHUNK = 160
GROUPS = CHUNK // LANES
NCHUNKS = SC_ROWS // CHUNK
KMAX = -(-NCHUNKS // NW)
HVECS = DIM // LANES

assert SC_ROWS % CHUNK == 0


def _sc_scan_body(query_hbm, mem_hbm, imp_hbm, encw_hbm, encb_hbm,
                  t_out, i_out,
                  qbuf, wbuf, bbuf, qebuf, membuf0, membuf1,
                  impbuf0, impbuf1, btbuf, bibuf, sem0, sem1):
    membufs = (membuf0, membuf1)
    impbufs = (impbuf0, impbuf1)
    sems = (sem0, sem1)
    cid = lax.axis_index("c")
    sid = lax.axis_index("s")
    wid = cid * NS + sid

    iota = lax.iota(jnp.int32, LANES)

    pltpu.sync_copy(query_hbm, qbuf)
    pltpu.sync_copy(encw_hbm, wbuf)
    pltpu.sync_copy(encb_hbm, bbuf)

    widx = [(iota + h * LANES) * DIM for h in range(HVECS)]

    def enc_step(_, carry):
        qskew, acc = carry
        qd = plsc.load_gather(qbuf, [qskew])
        acc = tuple(
            acc[h] + qd * plsc.load_gather(wbuf, [widx[h] + qskew])
            for h in range(HVECS)
        )
        return (qskew + 1) & (DIM - 1), acc

    acc0 = tuple(jnp.zeros((LANES,), jnp.float32) for _ in range(HVECS))
    _, acc = lax.fori_loop(0, DIM, enc_step, (iota, acc0), unroll=8)
    for h in range(HVECS):
        qebuf[pl.ds(h * LANES, LANES)] = acc[h] + bbuf[pl.ds(h * LANES, LANES)]

    btbuf[...] = jnp.full((LANES,), -jnp.inf, jnp.float32)
    bibuf[...] = jnp.zeros((LANES,), jnp.int32)

    bidx = [(iota + g * LANES) * DIM for g in range(GROUPS)]

    def _start(c, b):
        @pl.when(c < NCHUNKS)
        def _():
            base = SC_START + c * CHUNK
            pltpu.async_copy(mem_hbm.at[pl.ds(base * DIM, CHUNK * DIM)],
                             membufs[b], sems[b])
            pltpu.async_copy(imp_hbm.at[pl.ds(base, CHUNK)],
                             impbufs[b], sems[b])

    def _compute(c, b):
        @pl.when(c < NCHUNKS)
        def _():
            base = SC_START + c * CHUNK
            pltpu.make_async_copy(
                mem_hbm.at[pl.ds(base * DIM, CHUNK * DIM)],
                membufs[b], sems[b]).wait()
            pltpu.make_async_copy(
                imp_hbm.at[pl.ds(base, CHUNK)],
                impbufs[b], sems[b]).wait()

            def d_step(_, carry):
                qskew, dots, sqs = carry
                qd = plsc.load_gather(qebuf, [qskew])
                new_dots, new_sqs = [], []
                for g in range(GROUPS):
                    v = plsc.load_gather(membufs[b], [bidx[g] + qskew])
                    new_dots.append(dots[g] + v * qd)
                    new_sqs.append(sqs[g] + v * v)
                return ((qskew + 1) & (DIM - 1), tuple(new_dots),
                        tuple(new_sqs))

            z = tuple(jnp.zeros((LANES,), jnp.float32) for _ in range(GROUPS))
            _, dots, sqs = lax.fori_loop(0, DIM, d_step, (iota, z, z),
                                         unroll=8)

            bt = btbuf[...]
            bi = bibuf[...]
            for g in range(GROUPS):
                impv = impbufs[b][pl.ds(g * LANES, LANES)]
                sq = jnp.maximum(sqs[g], 1e-16)
                t = (dots[g] * jnp.abs(dots[g])) * (impv * impv) / sq
                ridx = base + g * LANES + iota
                upd = t > bt
                bt = jnp.where(upd, t, bt)
                bi = jnp.where(upd, ridx, bi)
            btbuf[...] = bt
            bibuf[...] = bi

            _start(c + 2 * NW, b)

    _start(wid, 0)
    _start(NW + wid, 1)

    def two_step(i, _):
        _compute(i * 2 * NW + wid, 0)
        _compute((i * 2 + 1) * NW + wid, 1)
        return 0

    lax.fori_loop(0, (KMAX + 1) // 2, two_step, 0)

    pltpu.sync_copy(btbuf, t_out.at[wid])
    pltpu.sync_copy(bibuf, i_out.at[wid])


_sc_scan = pl.kernel(
    _sc_scan_body,
    out_type=(jax.ShapeDtypeStruct((NW, LANES), jnp.float32),
              jax.ShapeDtypeStruct((NW, LANES), jnp.int32)),
    mesh=plsc.VectorSubcoreMesh(core_axis_name="c", subcore_axis_name="s",
                                num_cores=NC, num_subcores=NS),
    compiler_params=pltpu.CompilerParams(needs_layout_passes=False),
    scratch_types=[
        pltpu.VMEM((DIM,), jnp.float32),
        pltpu.VMEM((DIM * DIM,), jnp.float32),
        pltpu.VMEM((DIM,), jnp.float32),
        pltpu.VMEM((DIM,), jnp.float32),
        pltpu.VMEM((CHUNK * DIM,), jnp.float32),
        pltpu.VMEM((CHUNK * DIM,), jnp.float32),
        pltpu.VMEM((CHUNK,), jnp.float32),
        pltpu.VMEM((CHUNK,), jnp.float32),
        pltpu.VMEM((LANES,), jnp.float32),
        pltpu.VMEM((LANES,), jnp.int32),
        pltpu.SemaphoreType.DMA,
        pltpu.SemaphoreType.DMA,
    ],
)


TCW = 512


def _tc_scan_body(query_ref, encw_ref, encb_ref, imp_ref, x_ref,
                  t_out, i_out, qe_scr, bt_scr, bi_scr):
    step = pl.program_id(0)

    @pl.when(step == 0)
    def _():
        qe_scr[...] = lax.dot_general(
            query_ref[...], encw_ref[...], (((1,), (1,)), ((), ())),
            preferred_element_type=jnp.float32) + encb_ref[...]
        bt_scr[...] = jnp.full((1, TCW), -jnp.inf, jnp.float32)
        bi_scr[...] = jnp.zeros((1, TCW), jnp.int32)

    x = x_ref[...]
    qe = qe_scr[...]
    dot = lax.dot_general(qe, x, (((1,), (1,)), ((), ())),
                          preferred_element_type=jnp.float32)
    ones = jnp.ones((1, DIM), jnp.float32)
    sq = lax.dot_general(ones, x * x, (((1,), (1,)), ((), ())),
                         preferred_element_type=jnp.float32)
    imp = imp_ref[...]
    t = (dot * jnp.abs(dot)) * (imp * imp) / jnp.maximum(sq, 1e-16)
    rows = lax.broadcasted_iota(jnp.int32, (1, BLK), 1) + step * BLK

    w = BLK
    while w > TCW:
        w //= 2
        ta, tb = t[:, :w], t[:, w:2 * w]
        ra, rb = rows[:, :w], rows[:, w:2 * w]
        upd = tb > ta
        t = jnp.where(upd, tb, ta)
        rows = jnp.where(upd, rb, ra)
    bt = bt_scr[...]
    upd = t > bt
    bt_scr[...] = jnp.where(upd, t, bt)
    bi_scr[...] = jnp.where(upd, rows, bi_scr[...])

    @pl.when(step == NBLK - 1)
    def _():
        t_out[...] = bt_scr[...]
        i_out[...] = bi_scr[...]


def _tc_merge_decode(tsc_ref, isc_ref, ttc_ref, itc_ref, mem_ref,
                     w_ref, b_ref, out_ref, row_buf, sem):
    t = tsc_ref[...]
    idx = isc_ref[...]
    ttc = ttc_ref[...]
    itc = itc_ref[...]
    m = jnp.maximum(jnp.max(t), jnp.max(ttc))
    big = jnp.int32(2**31 - 1)
    r_sc = jnp.min(jnp.where(t == m, idx, big))
    r = jnp.minimum(r_sc, jnp.min(jnp.where(ttc == m, itc, big)))
    cp = pltpu.make_async_copy(mem_ref.at[pl.ds(r, 1)], row_buf, sem)
    cp.start()
    cp.wait()
    out_ref[...] = lax.dot_general(
        row_buf[...], w_ref[...], (((1,), (1,)), ((), ())),
        preferred_element_type=jnp.float32) + b_ref[...]


def kernel(query, mem_embeddings, importances, enc_W, enc_b, dec_W, dec_b):
    q2 = query.reshape(1, DIM)
    eb2 = enc_b.reshape(1, DIM)

    t_sc, i_sc = _sc_scan(query, mem_embeddings.reshape(-1), importances,
                          enc_W.reshape(-1), enc_b)

    t_tc, i_tc = pl.pallas_call(
        _tc_scan_body,
        grid=(NBLK,),
        out_shape=(jax.ShapeDtypeStruct((1, TCW), jnp.float32),
                   jax.ShapeDtypeStruct((1, TCW), jnp.int32)),
        in_specs=[
            pl.BlockSpec((1, DIM), lambda i: (0, 0)),
            pl.BlockSpec((DIM, DIM), lambda i: (0, 0)),
            pl.BlockSpec((1, DIM), lambda i: (0, 0)),
            pl.BlockSpec((1, BLK), lambda i: (0, i)),
            pl.BlockSpec((BLK, DIM), lambda i: (i, 0)),
        ],
        out_specs=(pl.BlockSpec((1, TCW), lambda i: (0, 0)),
                   pl.BlockSpec((1, TCW), lambda i: (0, 0))),
        scratch_shapes=[pltpu.VMEM((1, DIM), jnp.float32),
                        pltpu.VMEM((1, TCW), jnp.float32),
                        pltpu.VMEM((1, TCW), jnp.int32)],
    )(q2, enc_W, eb2, importances.reshape(1, CAP), mem_embeddings)

    out = pl.pallas_call(
        _tc_merge_decode,
        out_shape=jax.ShapeDtypeStruct((1, DIM), jnp.float32),
        in_specs=[
            pl.BlockSpec(memory_space=pltpu.VMEM),
            pl.BlockSpec(memory_space=pltpu.VMEM),
            pl.BlockSpec(memory_space=pltpu.VMEM),
            pl.BlockSpec(memory_space=pltpu.VMEM),
            pl.BlockSpec(memory_space=pl.ANY),
            pl.BlockSpec(memory_space=pltpu.VMEM),
            pl.BlockSpec(memory_space=pltpu.VMEM),
        ],
        out_specs=pl.BlockSpec(memory_space=pltpu.VMEM),
        scratch_shapes=[pltpu.VMEM((1, DIM), jnp.float32),
                        pltpu.SemaphoreType.DMA],
    )(t_sc, i_sc, t_tc, i_tc, mem_embeddings, dec_W, dec_b.reshape(1, DIM))
    return out.reshape(DIM)


# speedup vs baseline: 1.3113x; 1.3113x over previous
import jax
import jax.numpy as jnp
from jax import lax
from jax.experimental import pallas as pl
from jax.experimental.pallas import tpu as pltpu
from jax.experimental.pallas import tpu_sc as plsc

DIM = 128
CAP = 100000

BLK = 4096
NBLK = 20
TC_ROWS = NBLK * BLK
SC_START = TC_ROWS
SC_ROWS = CAP - TC_ROWS

NC = 1
NS = 16
NW = NC * NS
LANES = 16
CHUNK = 160
GROUPS = CHUNK // LANES
NCHUNKS = SC_ROWS // CHUNK
KMAX = -(-NCHUNKS // NW)
HVECS = DIM // LANES

assert SC_ROWS % CHUNK == 0


def _sc_scan_body(query_hbm, mem_hbm, imp_hbm, encw_hbm, encb_hbm,
                  t_out, i_out,
                  qbuf, wbuf, bbuf, qebuf, membuf0, membuf1,
                  impbuf0, impbuf1, btbuf, bibuf, sem0, sem1):
    membufs = (membuf0, membuf1)
    impbufs = (impbuf0, impbuf1)
    sems = (sem0, sem1)
    cid = lax.axis_index("c")
    sid = lax.axis_index("s")
    wid = cid * NS + sid

    iota = lax.iota(jnp.int32, LANES)

    pltpu.sync_copy(query_hbm, qbuf)
    pltpu.sync_copy(encw_hbm, wbuf)
    pltpu.sync_copy(encb_hbm, bbuf)

    widx = [(iota + h * LANES) * DIM for h in range(HVECS)]

    def enc_step(_, carry):
        qskew, acc = carry
        qd = plsc.load_gather(qbuf, [qskew])
        acc = tuple(
            acc[h] + qd * plsc.load_gather(wbuf, [widx[h] + qskew])
            for h in range(HVECS)
        )
        return (qskew + 1) & (DIM - 1), acc

    acc0 = tuple(jnp.zeros((LANES,), jnp.float32) for _ in range(HVECS))
    _, acc = lax.fori_loop(0, DIM, enc_step, (iota, acc0), unroll=8)
    for h in range(HVECS):
        qebuf[pl.ds(h * LANES, LANES)] = acc[h] + bbuf[pl.ds(h * LANES, LANES)]

    btbuf[...] = jnp.full((LANES,), -jnp.inf, jnp.float32)
    bibuf[...] = jnp.zeros((LANES,), jnp.int32)

    bidx = [(iota + g * LANES) * DIM for g in range(GROUPS)]

    def _start(c, b):
        @pl.when(c < NCHUNKS)
        def _():
            base = SC_START + c * CHUNK
            pltpu.async_copy(mem_hbm.at[pl.ds(base * DIM, CHUNK * DIM)],
                             membufs[b], sems[b])
            pltpu.async_copy(imp_hbm.at[pl.ds(base, CHUNK)],
                             impbufs[b], sems[b])

    def _compute(c, b):
        @pl.when(c < NCHUNKS)
        def _():
            base = SC_START + c * CHUNK
            pltpu.make_async_copy(
                mem_hbm.at[pl.ds(base * DIM, CHUNK * DIM)],
                membufs[b], sems[b]).wait()
            pltpu.make_async_copy(
                imp_hbm.at[pl.ds(base, CHUNK)],
                impbufs[b], sems[b]).wait()

            def d_step(_, carry):
                qskew, dots, sqs = carry
                qd = plsc.load_gather(qebuf, [qskew])
                new_dots, new_sqs = [], []
                for g in range(GROUPS):
                    v = plsc.load_gather(membufs[b], [bidx[g] + qskew])
                    new_dots.append(dots[g] + v * qd)
                    new_sqs.append(sqs[g] + v * v)
                return ((qskew + 1) & (DIM - 1), tuple(new_dots),
                        tuple(new_sqs))

            z = tuple(jnp.zeros((LANES,), jnp.float32) for _ in range(GROUPS))
            _, dots, sqs = lax.fori_loop(0, DIM, d_step, (iota, z, z),
                                         unroll=8)

            bt = btbuf[...]
            bi = bibuf[...]
            for g in range(GROUPS):
                impv = impbufs[b][pl.ds(g * LANES, LANES)]
                sq = jnp.maximum(sqs[g], 1e-16)
                t = (dots[g] * jnp.abs(dots[g])) * (impv * impv) / sq
                ridx = base + g * LANES + iota
                upd = t > bt
                bt = jnp.where(upd, t, bt)
                bi = jnp.where(upd, ridx, bi)
            btbuf[...] = bt
            bibuf[...] = bi

            _start(c + 2 * NW, b)

    _start(wid, 0)
    _start(NW + wid, 1)

    def two_step(i, _):
        _compute(i * 2 * NW + wid, 0)
        _compute((i * 2 + 1) * NW + wid, 1)
        return 0

    lax.fori_loop(0, (KMAX + 1) // 2, two_step, 0)

    pltpu.sync_copy(btbuf, t_out.at[wid])
    pltpu.sync_copy(bibuf, i_out.at[wid])


_sc_scan = pl.kernel(
    _sc_scan_body,
    out_type=(jax.ShapeDtypeStruct((NW, LANES), jnp.float32),
              jax.ShapeDtypeStruct((NW, LANES), jnp.int32)),
    mesh=plsc.VectorSubcoreMesh(core_axis_name="c", subcore_axis_name="s",
                                num_cores=NC, num_subcores=NS),
    compiler_params=pltpu.CompilerParams(needs_layout_passes=False),
    scratch_types=[
        pltpu.VMEM((DIM,), jnp.float32),
        pltpu.VMEM((DIM * DIM,), jnp.float32),
        pltpu.VMEM((DIM,), jnp.float32),
        pltpu.VMEM((DIM,), jnp.float32),
        pltpu.VMEM((CHUNK * DIM,), jnp.float32),
        pltpu.VMEM((CHUNK * DIM,), jnp.float32),
        pltpu.VMEM((CHUNK,), jnp.float32),
        pltpu.VMEM((CHUNK,), jnp.float32),
        pltpu.VMEM((LANES,), jnp.float32),
        pltpu.VMEM((LANES,), jnp.int32),
        pltpu.SemaphoreType.DMA,
        pltpu.SemaphoreType.DMA,
    ],
)


TCW = 512


def _tc_scan_body(query_ref, encw_ref, encb_ref, imp_ref, x_ref,
                  t_out, i_out, qe_scr, bt_scr, bi_scr):
    step = pl.program_id(0)

    @pl.when(step == 0)
    def _():
        qe_scr[...] = lax.dot_general(
            query_ref[...], encw_ref[...], (((1,), (1,)), ((), ())),
            preferred_element_type=jnp.float32) + encb_ref[...]
        bt_scr[...] = jnp.full((1, TCW), -jnp.inf, jnp.float32)
        bi_scr[...] = jnp.zeros((1, TCW), jnp.int32)

    x = x_ref[...]
    qe = qe_scr[...]
    dot = lax.dot_general(qe, x, (((1,), (1,)), ((), ())),
                          preferred_element_type=jnp.float32)
    ones = jnp.ones((1, DIM), jnp.float32)
    sq = lax.dot_general(ones, x * x, (((1,), (1,)), ((), ())),
                         preferred_element_type=jnp.float32)
    imp = imp_ref[...]
    t = (dot * jnp.abs(dot)) * (imp * imp) / jnp.maximum(sq, 1e-16)
    rows = lax.broadcasted_iota(jnp.int32, (1, BLK), 1) + step * BLK

    w = BLK
    while w > TCW:
        w //= 2
        ta, tb = t[:, :w], t[:, w:2 * w]
        ra, rb = rows[:, :w], rows[:, w:2 * w]
        upd = tb > ta
        t = jnp.where(upd, tb, ta)
        rows = jnp.where(upd, rb, ra)
    bt = bt_scr[...]
    upd = t > bt
    bt_scr[...] = jnp.where(upd, t, bt)
    bi_scr[...] = jnp.where(upd, rows, bi_scr[...])

    @pl.when(step == NBLK - 1)
    def _():
        t_out[...] = bt_scr[...]
        i_out[...] = bi_scr[...]


def _tc_merge_decode(tsc_ref, isc_ref, ttc_ref, itc_ref, mem_ref,
                     w_ref, b_ref, out_ref, row_buf, sem):
    t = tsc_ref[...]
    idx = isc_ref[...]
    ttc = ttc_ref[...]
    itc = itc_ref[...]
    m = jnp.maximum(jnp.max(t), jnp.max(ttc))
    big = jnp.int32(2**31 - 1)
    r_sc = jnp.min(jnp.where(t == m, idx, big))
    r = jnp.minimum(r_sc, jnp.min(jnp.where(ttc == m, itc, big)))
    cp = pltpu.make_async_copy(mem_ref.at[pl.ds(r, 1)], row_buf, sem)
    cp.start()
    cp.wait()
    out_ref[...] = lax.dot_general(
        row_buf[...], w_ref[...], (((1,), (1,)), ((), ())),
        preferred_element_type=jnp.float32) + b_ref[...]


def kernel(query, mem_embeddings, importances, enc_W, enc_b, dec_W, dec_b):
    q2 = query.reshape(1, DIM)
    eb2 = enc_b.reshape(1, DIM)

    t_sc, i_sc = _sc_scan(query, mem_embeddings.reshape(-1), importances,
                          enc_W.reshape(-1), enc_b)

    t_tc, i_tc = pl.pallas_call(
        _tc_scan_body,
        grid=(NBLK,),
        out_shape=(jax.ShapeDtypeStruct((1, TCW), jnp.float32),
                   jax.ShapeDtypeStruct((1, TCW), jnp.int32)),
        in_specs=[
            pl.BlockSpec((1, DIM), lambda i: (0, 0)),
            pl.BlockSpec((DIM, DIM), lambda i: (0, 0)),
            pl.BlockSpec((1, DIM), lambda i: (0, 0)),
            pl.BlockSpec((1, BLK), lambda i: (0, i)),
            pl.BlockSpec((BLK, DIM), lambda i: (i, 0)),
        ],
        out_specs=(pl.BlockSpec((1, TCW), lambda i: (0, 0)),
                   pl.BlockSpec((1, TCW), lambda i: (0, 0))),
        scratch_shapes=[pltpu.VMEM((1, DIM), jnp.float32),
                        pltpu.VMEM((1, TCW), jnp.float32),
                        pltpu.VMEM((1, TCW), jnp.int32)],
    )(q2, enc_W, eb2, importances.reshape(1, CAP), mem_embeddings)

    out = pl.pallas_call(
        _tc_merge_decode,
        out_shape=jax.ShapeDtypeStruct((1, DIM), jnp.float32),
        in_specs=[
            pl.BlockSpec(memory_space=pltpu.VMEM),
            pl.BlockSpec(memory_space=pltpu.VMEM),
            pl.BlockSpec(memory_space=pltpu.VMEM),
            pl.BlockSpec(memory_space=pltpu.VMEM),
            pl.BlockSpec(memory_space=pl.ANY),
            pl.BlockSpec(memory_space=pltpu.VMEM),
            pl.BlockSpec(memory_space=pltpu.VMEM),
        ],
        out_specs=pl.BlockSpec(memory_space=pltpu.VMEM),
        scratch_shapes=[pltpu.VMEM((1, DIM), jnp.float32),
                        pltpu.SemaphoreType.DMA],
    )(t_sc, i_sc, t_tc, i_tc, mem_embeddings, dec_W, dec_b.reshape(1, DIM))
    return out.reshape(DIM)
